# baseline (speedup 1.0000x reference)
import functools

import jax
import jax.numpy as jnp
from jax import lax
from jax.experimental import pallas as pl
from jax.experimental.pallas import tpu as pltpu
from jax.experimental.pallas import tpu_sc as plsc

E = 8
D = 2048
DH = D // 2
DHW = DH // 2
DFF = 1408
T = 8192
TK = 2 * T
BS = 512
NB = TK // BS + E
NBP = NB + 8
P = NB * BS
RT = 1024

NC = 2
NS = 16
NW = NC * NS
SROWS = 64
WROWS = SROWS // NW


def _pack_pair(xf32):
    h = xf32.shape[1] // 2
    lo = lax.bitcast_convert_type(
        xf32[:, :h].astype(jnp.bfloat16).astype(jnp.float32), jnp.int32)
    hi = lax.bitcast_convert_type(
        xf32[:, h:].astype(jnp.bfloat16).astype(jnp.float32), jnp.int32)
    return lax.shift_right_logical(lo, 16) | (hi & (-65536))


def _unpack2(w):
    lo = lax.bitcast_convert_type(lax.shift_left(w, 16), jnp.float32)
    hi = lax.bitcast_convert_type(w & (-65536), jnp.float32)
    return lo, hi


def _router_body(x_ref, gw_ref, aux_ref, xa_ref, xb_ref):
    x = x_ref[...]
    logits = lax.dot_general(x, gw_ref[...], (((1,), (0,)), ((), ())),
                             preferred_element_type=jnp.float32)
    p = jax.nn.softmax(logits, axis=-1)
    lane = lax.broadcasted_iota(jnp.int32, p.shape, 1)
    m1 = jnp.max(p, axis=-1, keepdims=True)
    idx1 = jnp.min(jnp.where(p == m1, lane, E), axis=-1, keepdims=True)
    pex = jnp.where(lane == idx1, -1.0, p)
    m2 = jnp.max(pex, axis=-1, keepdims=True)
    idx2 = jnp.min(jnp.where(pex == m2, lane, E), axis=-1, keepdims=True)
    denom = m1 + m2
    swap = idx2 < idx1
    elo = jnp.where(swap, idx2, idx1).astype(jnp.float32)
    ehi = jnp.where(swap, idx1, idx2).astype(jnp.float32)
    wlo = jnp.where(swap, m2, m1) / denom
    whi = jnp.where(swap, m1, m2) / denom
    aux_ref[...] = (jnp.where(lane == 0, elo, 0.0)
                    + jnp.where(lane == 1, ehi, 0.0)
                    + jnp.where(lane == 2, wlo, 0.0)
                    + jnp.where(lane == 3, whi, 0.0))
    xa_ref[...] = _pack_pair(x[:, :DH])
    xb_ref[...] = _pack_pair(x[:, DH:])


def _slots_body(aux_ref, slo_ref, shi_ref, bexp_ref):
    aux = aux_ref[...]
    lane = lax.broadcasted_iota(jnp.int32, (T, E), 1)
    oh_lo = (lane == aux[:, 0:1].astype(jnp.int32)).astype(jnp.float32)
    oh_hi = (lane == aux[:, 1:2].astype(jnp.int32)).astype(jnp.float32)
    v_lo = oh_lo.reshape(SROWS, 128, E)
    v_hi = oh_hi.reshape(SROWS, 128, E)

    r_i = lax.broadcasted_iota(jnp.int32, (SROWS, 128, 128), 1)
    r_j = lax.broadcasted_iota(jnp.int32, (SROWS, 128, 128), 2)
    ltri = (r_j < r_i).astype(jnp.float32)

    def pancum(v):
        return lax.dot_general(ltri, v, (((2,), (1,)), ((0,), (0,))),
                               preferred_element_type=jnp.float32)

    w_lo, w_hi = pancum(v_lo), pancum(v_hi)
    s_lo = jnp.sum(v_lo, axis=1)
    s_hi = jnp.sum(v_hi, axis=1)
    p_i = lax.broadcasted_iota(jnp.int32, (SROWS, SROWS), 0)
    p_j = lax.broadcasted_iota(jnp.int32, (SROWS, SROWS), 1)
    l64 = (p_j < p_i).astype(jnp.float32)
    off_lo = lax.dot_general(l64, s_lo, (((1,), (0,)), ((), ())),
                             preferred_element_type=jnp.float32)
    off_hi = lax.dot_general(l64, s_hi, (((1,), (0,)), ((), ())),
                             preferred_element_type=jnp.float32)
    tot_lo = jnp.sum(s_lo, axis=0, keepdims=True)
    tot_hi = jnp.sum(s_hi, axis=0, keepdims=True)

    counts = (tot_lo + tot_hi).astype(jnp.int32)
    padded = (((counts + BS - 1) // BS) * BS).astype(jnp.float32)
    e_i = lax.broadcasted_iota(jnp.int32, (E, E), 0)
    e_j = lax.broadcasted_iota(jnp.int32, (E, E), 1)
    m8 = (e_i < e_j).astype(jnp.float32)
    gstart = lax.dot_general(padded, m8, (((1,), (0,)), ((), ())),
                             preferred_element_type=jnp.float32)

    c_lo = w_lo + off_lo.reshape(SROWS, 1, E)
    c_hi = w_hi + off_hi.reshape(SROWS, 1, E) + tot_lo.reshape(1, 1, E)
    base = gstart.reshape(1, 1, E)
    slot_lo = jnp.sum((c_lo + base) * v_lo, axis=2)
    slot_hi = jnp.sum((c_hi + base) * v_hi, axis=2)
    slo_ref[...] = slot_lo.astype(jnp.int32)
    shi_ref[...] = slot_hi.astype(jnp.int32)

    tot_pad = jnp.sum(padded, axis=1, keepdims=True)
    b_i = (lax.broadcasted_iota(jnp.int32, (NBP, E), 0) * BS).astype(jnp.float32)
    b_c = jnp.minimum(b_i, tot_pad - BS)
    nle = jnp.sum((gstart <= b_c).astype(jnp.float32), axis=1,
                  keepdims=True) - 1.0
    row = lax.broadcasted_iota(jnp.int32, (NBP, 1), 0)
    val = jnp.where(row < NB, nle, tot_pad / BS)
    bexp_ref[...] = jnp.broadcast_to(val, (NBP, 128))


def _dispatch_body(xa_hbm, xb_hbm, slo_hbm, shi_hbm, oa_hbm, ob_hbm,
                   slot_v, buf_v):
    wid = lax.axis_index("s") * NC + lax.axis_index("c")
    for slot_hbm in (slo_hbm, shi_hbm):
        for r in range(WROWS):
            tok0 = (wid * WROWS + r) * 128
            pltpu.sync_copy(slot_hbm.at[pl.ds(tok0, 128)], slot_v)
            pltpu.sync_copy(xa_hbm.at[pl.ds(tok0, 128)], buf_v)
            pltpu.sync_copy(buf_v, oa_hbm.at[slot_v])
            pltpu.sync_copy(xb_hbm.at[pl.ds(tok0, 128)], buf_v)
            pltpu.sync_copy(buf_v, ob_hbm.at[slot_v])


def _mlp_body(bexp_ref, xa_ref, xb_ref, gup_ref, dwn_ref, ya_ref, yb_ref,
              dbf_ref):
    b = pl.program_id(0)

    @pl.when(b < bexp_ref[NB])
    def _():
        changed = jnp.logical_or(
            b == 0, bexp_ref[b] != bexp_ref[jnp.maximum(b - 1, 0)])

        @pl.when(changed)
        def _():
            dbf_ref[...] = dwn_ref[0].astype(jnp.bfloat16)

        _mlp_compute(xa_ref, xb_ref, gup_ref, dbf_ref, ya_ref, yb_ref)


def _mlp_compute(xa_ref, xb_ref, gup_ref, dbf_ref, ya_ref, yb_ref):
    la, ha = _unpack2(xa_ref[...])
    lb, hb = _unpack2(xb_ref[...])
    x = jnp.concatenate([la, ha, lb, hb], axis=1).astype(jnp.bfloat16)
    gg = lax.dot_general(x, gup_ref[0, :, :DFF], (((1,), (0,)), ((), ())),
                         preferred_element_type=jnp.float32)
    gu = lax.dot_general(x, gup_ref[0, :, DFF:], (((1,), (0,)), ((), ())),
                         preferred_element_type=jnp.float32)
    a = (jax.nn.silu(gg) * gu).astype(jnp.bfloat16)
    y = lax.dot_general(a, dbf_ref[...], (((1,), (0,)), ((), ())),
                        preferred_element_type=jnp.float32)
    ya_ref[...] = _pack_pair(y[:, :DH])
    yb_ref[...] = _pack_pair(y[:, DH:])


def _gather_body(ya_hbm, yb_hbm, slo_hbm, shi_hbm,
                 la_hbm, lb_hbm, ha_hbm, hb_hbm, slot_v, buf_v):
    wid = lax.axis_index("s") * NC + lax.axis_index("c")
    for slot_hbm, da_hbm, db_hbm in ((slo_hbm, la_hbm, lb_hbm),
                                     (shi_hbm, ha_hbm, hb_hbm)):
        for r in range(WROWS):
            tok0 = (wid * WROWS + r) * 128
            pltpu.sync_copy(slot_hbm.at[pl.ds(tok0, 128)], slot_v)
            pltpu.sync_copy(ya_hbm.at[slot_v], buf_v)
            pltpu.sync_copy(buf_v, da_hbm.at[pl.ds(tok0, 128)])
            pltpu.sync_copy(yb_hbm.at[slot_v], buf_v)
            pltpu.sync_copy(buf_v, db_hbm.at[pl.ds(tok0, 128)])


def _combine_body(aux_ref, la_ref, lb_ref, ha_ref, hb_ref, out_ref):
    aux = aux_ref[...]
    lane = lax.broadcasted_iota(jnp.int32, aux.shape, 1)
    wlo = jnp.sum(jnp.where(lane == 2, aux, 0.0), axis=1, keepdims=True)
    whi = jnp.sum(jnp.where(lane == 3, aux, 0.0), axis=1, keepdims=True)
    l0, l1 = _unpack2(la_ref[...])
    l2, l3 = _unpack2(lb_ref[...])
    h0, h1 = _unpack2(ha_ref[...])
    h2, h3 = _unpack2(hb_ref[...])
    for i, (lv, hv) in enumerate(((l0, h0), (l1, h1), (l2, h2), (l3, h3))):
        out_ref[:, i * DHW:(i + 1) * DHW] = wlo * lv + whi * hv


def kernel(hidden_states, gate_w, gate_up_w, down_w):
    f32 = jnp.float32
    bf16 = jnp.bfloat16

    aux, xa, xb = pl.pallas_call(
        _router_body,
        grid=(T // RT,),
        in_specs=[
            pl.BlockSpec((RT, D), lambda t: (t, 0)),
            pl.BlockSpec((D, E), lambda t: (0, 0)),
        ],
        out_specs=[
            pl.BlockSpec((RT, E), lambda t: (t, 0)),
            pl.BlockSpec((RT, DHW), lambda t: (t, 0)),
            pl.BlockSpec((RT, DHW), lambda t: (t, 0)),
        ],
        out_shape=[
            jax.ShapeDtypeStruct((T, E), f32),
            jax.ShapeDtypeStruct((T, DHW), jnp.int32),
            jax.ShapeDtypeStruct((T, DHW), jnp.int32),
        ],
    )(hidden_states, gate_w)

    slo3, shi3, bexpf = pl.pallas_call(
        _slots_body,
        out_shape=[
            jax.ShapeDtypeStruct((SROWS, 128), jnp.int32),
            jax.ShapeDtypeStruct((SROWS, 128), jnp.int32),
            jax.ShapeDtypeStruct((NBP, 128), f32),
        ],
    )(aux)
    slo1 = slo3.reshape(T)
    shi1 = shi3.reshape(T)
    bexp = bexpf[:, 0].astype(jnp.int32)

    mesh = plsc.VectorSubcoreMesh(core_axis_name="c", subcore_axis_name="s")

    dispatch = pl.kernel(
        _dispatch_body,
        out_type=[
            jax.ShapeDtypeStruct((P, DHW), jnp.int32),
            jax.ShapeDtypeStruct((P, DHW), jnp.int32),
        ],
        mesh=mesh,
        scratch_types=[
            pltpu.VMEM((128,), jnp.int32),
            pltpu.VMEM((128, DHW), jnp.int32),
        ],
    )
    xsa3, xsb3 = dispatch(xa, xb, slo1, shi1)

    gup_bf = gate_up_w.astype(bf16)

    ya, yb = pl.pallas_call(
        _mlp_body,
        grid_spec=pltpu.PrefetchScalarGridSpec(
            num_scalar_prefetch=1,
            grid=(NB,),
            in_specs=[
                pl.BlockSpec((BS, DHW), lambda b, be: (b, 0)),
                pl.BlockSpec((BS, DHW), lambda b, be: (b, 0)),
                pl.BlockSpec((1, D, 2 * DFF), lambda b, be: (be[b], 0, 0)),
                pl.BlockSpec((1, DFF, D), lambda b, be: (be[b], 0, 0)),
            ],
            out_specs=[
                pl.BlockSpec((BS, DHW), lambda b, be: (b, 0)),
                pl.BlockSpec((BS, DHW), lambda b, be: (b, 0)),
            ],
            scratch_shapes=[pltpu.VMEM((DFF, D), jnp.bfloat16)],
        ),
        out_shape=[
            jax.ShapeDtypeStruct((P, DHW), jnp.int32),
            jax.ShapeDtypeStruct((P, DHW), jnp.int32),
        ],
        compiler_params=pltpu.CompilerParams(
            dimension_semantics=("arbitrary",),
            vmem_limit_bytes=100 * 1024 * 1024,
        ),
    )(bexp, xsa3, xsb3, gup_bf, down_w)

    gather = pl.kernel(
        _gather_body,
        out_type=[
            jax.ShapeDtypeStruct((T, DHW), jnp.int32),
            jax.ShapeDtypeStruct((T, DHW), jnp.int32),
            jax.ShapeDtypeStruct((T, DHW), jnp.int32),
            jax.ShapeDtypeStruct((T, DHW), jnp.int32),
        ],
        mesh=mesh,
        scratch_types=[
            pltpu.VMEM((128,), jnp.int32),
            pltpu.VMEM((128, DHW), jnp.int32),
        ],
    )
    la3, lb3, ha3, hb3 = gather(ya, yb, slo1, shi1)

    out = pl.pallas_call(
        _combine_body,
        grid=(T // RT,),
        in_specs=[
            pl.BlockSpec((RT, E), lambda t: (t, 0)),
            pl.BlockSpec((RT, DHW), lambda t: (t, 0)),
            pl.BlockSpec((RT, DHW), lambda t: (t, 0)),
            pl.BlockSpec((RT, DHW), lambda t: (t, 0)),
            pl.BlockSpec((RT, DHW), lambda t: (t, 0)),
        ],
        out_specs=pl.BlockSpec((RT, D), lambda t: (t, 0)),
        out_shape=jax.ShapeDtypeStruct((T, D), f32),
    )(aux, la3, lb3, ha3, hb3)
    return out


# speedup vs baseline: 1.0079x; 1.0079x over previous
import functools

import jax
import jax.numpy as jnp
from jax import lax
from jax.experimental import pallas as pl
from jax.experimental.pallas import tpu as pltpu
from jax.experimental.pallas import tpu_sc as plsc

E = 8
D = 2048
DH = D // 2
DHW = DH // 2
DFF = 1408
T = 8192
TK = 2 * T
BS = 512
NB = TK // BS + E
NBP = NB + 8
P = NB * BS
RT = 1024

NC = 2
NS = 16
NW = NC * NS
SROWS = 64
WROWS = SROWS // NW


def _pack_pair(xf32):
    h = xf32.shape[1] // 2
    lo = lax.bitcast_convert_type(
        xf32[:, :h].astype(jnp.bfloat16).astype(jnp.float32), jnp.int32)
    hi = lax.bitcast_convert_type(
        xf32[:, h:].astype(jnp.bfloat16).astype(jnp.float32), jnp.int32)
    return lax.shift_right_logical(lo, 16) | (hi & (-65536))


def _unpack2(w):
    lo = lax.bitcast_convert_type(lax.shift_left(w, 16), jnp.float32)
    hi = lax.bitcast_convert_type(w & (-65536), jnp.float32)
    return lo, hi


def _router_body(x_ref, gw_ref, aux_ref, xa_ref, xb_ref):
    x = x_ref[...]
    logits = lax.dot_general(x, gw_ref[...], (((1,), (0,)), ((), ())),
                             preferred_element_type=jnp.float32)
    p = jax.nn.softmax(logits, axis=-1)
    lane = lax.broadcasted_iota(jnp.int32, p.shape, 1)
    m1 = jnp.max(p, axis=-1, keepdims=True)
    idx1 = jnp.min(jnp.where(p == m1, lane, E), axis=-1, keepdims=True)
    pex = jnp.where(lane == idx1, -1.0, p)
    m2 = jnp.max(pex, axis=-1, keepdims=True)
    idx2 = jnp.min(jnp.where(pex == m2, lane, E), axis=-1, keepdims=True)
    denom = m1 + m2
    swap = idx2 < idx1
    elo = jnp.where(swap, idx2, idx1).astype(jnp.float32)
    ehi = jnp.where(swap, idx1, idx2).astype(jnp.float32)
    wlo = jnp.where(swap, m2, m1) / denom
    whi = jnp.where(swap, m1, m2) / denom
    aux_ref[...] = (jnp.where(lane == 0, elo, 0.0)
                    + jnp.where(lane == 1, ehi, 0.0)
                    + jnp.where(lane == 2, wlo, 0.0)
                    + jnp.where(lane == 3, whi, 0.0))
    xa_ref[...] = _pack_pair(x[:, :DH])
    xb_ref[...] = _pack_pair(x[:, DH:])


def _slots_body(aux_ref, slo_ref, shi_ref, bexp_ref):
    aux = aux_ref[...]
    lane = lax.broadcasted_iota(jnp.int32, (T, E), 1)
    oh_lo = (lane == aux[:, 0:1].astype(jnp.int32)).astype(jnp.float32)
    oh_hi = (lane == aux[:, 1:2].astype(jnp.int32)).astype(jnp.float32)
    v_lo = oh_lo.reshape(SROWS, 128, E)
    v_hi = oh_hi.reshape(SROWS, 128, E)

    r_i = lax.broadcasted_iota(jnp.int32, (SROWS, 128, 128), 1)
    r_j = lax.broadcasted_iota(jnp.int32, (SROWS, 128, 128), 2)
    ltri = (r_j < r_i).astype(jnp.float32)

    def pancum(v):
        return lax.dot_general(ltri, v, (((2,), (1,)), ((0,), (0,))),
                               preferred_element_type=jnp.float32)

    w_lo, w_hi = pancum(v_lo), pancum(v_hi)
    s_lo = jnp.sum(v_lo, axis=1)
    s_hi = jnp.sum(v_hi, axis=1)
    p_i = lax.broadcasted_iota(jnp.int32, (SROWS, SROWS), 0)
    p_j = lax.broadcasted_iota(jnp.int32, (SROWS, SROWS), 1)
    l64 = (p_j < p_i).astype(jnp.float32)
    off_lo = lax.dot_general(l64, s_lo, (((1,), (0,)), ((), ())),
                             preferred_element_type=jnp.float32)
    off_hi = lax.dot_general(l64, s_hi, (((1,), (0,)), ((), ())),
                             preferred_element_type=jnp.float32)
    tot_lo = jnp.sum(s_lo, axis=0, keepdims=True)
    tot_hi = jnp.sum(s_hi, axis=0, keepdims=True)

    counts = (tot_lo + tot_hi).astype(jnp.int32)
    padded = (((counts + BS - 1) // BS) * BS).astype(jnp.float32)
    e_i = lax.broadcasted_iota(jnp.int32, (E, E), 0)
    e_j = lax.broadcasted_iota(jnp.int32, (E, E), 1)
    m8 = (e_i < e_j).astype(jnp.float32)
    gstart = lax.dot_general(padded, m8, (((1,), (0,)), ((), ())),
                             preferred_element_type=jnp.float32)

    c_lo = w_lo + off_lo.reshape(SROWS, 1, E)
    c_hi = w_hi + off_hi.reshape(SROWS, 1, E) + tot_lo.reshape(1, 1, E)
    base = gstart.reshape(1, 1, E)
    slot_lo = jnp.sum((c_lo + base) * v_lo, axis=2)
    slot_hi = jnp.sum((c_hi + base) * v_hi, axis=2)
    slo_ref[...] = slot_lo.astype(jnp.int32)
    shi_ref[...] = slot_hi.astype(jnp.int32)

    tot_pad = jnp.sum(padded, axis=1, keepdims=True)
    b_i = (lax.broadcasted_iota(jnp.int32, (NBP, E), 0) * BS).astype(jnp.float32)
    b_c = jnp.minimum(b_i, tot_pad - BS)
    nle = jnp.sum((gstart <= b_c).astype(jnp.float32), axis=1,
                  keepdims=True) - 1.0
    row = lax.broadcasted_iota(jnp.int32, (NBP, 1), 0)
    val = jnp.where(row < NB, nle, tot_pad / BS)
    bexp_ref[...] = jnp.broadcast_to(val, (NBP, 128))


def _dispatch_body(xa_hbm, xb_hbm, slo_hbm, shi_hbm, oa_hbm, ob_hbm,
                   slot_v, buf_v):
    wid = lax.axis_index("s") * NC + lax.axis_index("c")
    for slot_hbm in (slo_hbm, shi_hbm):
        for r in range(WROWS):
            tok0 = (wid * WROWS + r) * 128
            pltpu.sync_copy(slot_hbm.at[pl.ds(tok0, 128)], slot_v)
            pltpu.sync_copy(xa_hbm.at[pl.ds(tok0, 128)], buf_v)
            pltpu.sync_copy(buf_v, oa_hbm.at[slot_v])
            pltpu.sync_copy(xb_hbm.at[pl.ds(tok0, 128)], buf_v)
            pltpu.sync_copy(buf_v, ob_hbm.at[slot_v])


def _mlp_body(bexp_ref, xa_ref, xb_ref, gup_ref, dwn_ref, ya_ref, yb_ref):
    @pl.when(pl.program_id(0) < bexp_ref[NB])
    def _():
        _mlp_compute(xa_ref, xb_ref, gup_ref, dwn_ref, ya_ref, yb_ref)


def _mlp_compute(xa_ref, xb_ref, gup_ref, dwn_ref, ya_ref, yb_ref):
    la, ha = _unpack2(xa_ref[...])
    lb, hb = _unpack2(xb_ref[...])
    x = jnp.concatenate([la, ha, lb, hb], axis=1).astype(jnp.bfloat16)
    gg = lax.dot_general(x, gup_ref[0, :, :DFF], (((1,), (0,)), ((), ())),
                         preferred_element_type=jnp.float32)
    gu = lax.dot_general(x, gup_ref[0, :, DFF:], (((1,), (0,)), ((), ())),
                         preferred_element_type=jnp.float32)
    a = (jax.nn.silu(gg) * gu).astype(jnp.bfloat16)
    y = lax.dot_general(a, dwn_ref[0].astype(jnp.bfloat16),
                        (((1,), (0,)), ((), ())),
                        preferred_element_type=jnp.float32)
    ya_ref[...] = _pack_pair(y[:, :DH])
    yb_ref[...] = _pack_pair(y[:, DH:])


def _gather_body(ya_hbm, yb_hbm, slo_hbm, shi_hbm,
                 la_hbm, lb_hbm, ha_hbm, hb_hbm,
                 i0, i1, i2, i3, i4, i5, i6, i7,
                 buf0, buf1, sg0, sg1, sw0, sw1):
    wid = lax.axis_index("s") * NC + lax.axis_index("c")
    idx_refs = (i0, i1, i2, i3, i4, i5, i6, i7)
    items = []
    k = 0
    for slot_hbm, da_hbm, db_hbm in ((slo_hbm, la_hbm, lb_hbm),
                                     (shi_hbm, ha_hbm, hb_hbm)):
        for r in range(WROWS):
            for c in range(2):
                tok = (wid * WROWS + r) * 128 + c * 64
                pltpu.sync_copy(slot_hbm.at[pl.ds(tok, 64)], idx_refs[k])
                items.append((idx_refs[k], ya_hbm, da_hbm, tok))
                items.append((idx_refs[k], yb_hbm, db_hbm, tok))
                k += 1
    bufs, sgs, sws = (buf0, buf1), (sg0, sg1), (sw0, sw1)
    gops = [None, None]
    wops = [None, None]
    n = len(items)
    for i in range(n + 1):
        if i < n:
            if i >= 2:
                wops[i % 2].wait()
            idx, src, _, _ = items[i]
            gops[i % 2] = pltpu.make_async_copy(src.at[idx], bufs[i % 2],
                                                sgs[i % 2])
            gops[i % 2].start()
        if i >= 1:
            j = i - 1
            gops[j % 2].wait()
            _, _, dst, tok = items[j]
            wops[j % 2] = pltpu.make_async_copy(
                bufs[j % 2], dst.at[pl.ds(tok, 64)], sws[j % 2])
            wops[j % 2].start()
    wops[(n - 1) % 2].wait()
    wops[(n - 2) % 2].wait()


def _combine_body(aux_ref, la_ref, lb_ref, ha_ref, hb_ref, out_ref):
    aux = aux_ref[...]
    lane = lax.broadcasted_iota(jnp.int32, aux.shape, 1)
    wlo = jnp.sum(jnp.where(lane == 2, aux, 0.0), axis=1, keepdims=True)
    whi = jnp.sum(jnp.where(lane == 3, aux, 0.0), axis=1, keepdims=True)
    l0, l1 = _unpack2(la_ref[...])
    l2, l3 = _unpack2(lb_ref[...])
    h0, h1 = _unpack2(ha_ref[...])
    h2, h3 = _unpack2(hb_ref[...])
    for i, (lv, hv) in enumerate(((l0, h0), (l1, h1), (l2, h2), (l3, h3))):
        out_ref[:, i * DHW:(i + 1) * DHW] = wlo * lv + whi * hv


def kernel(hidden_states, gate_w, gate_up_w, down_w):
    f32 = jnp.float32
    bf16 = jnp.bfloat16

    aux, xa, xb = pl.pallas_call(
        _router_body,
        grid=(T // RT,),
        in_specs=[
            pl.BlockSpec((RT, D), lambda t: (t, 0)),
            pl.BlockSpec((D, E), lambda t: (0, 0)),
        ],
        out_specs=[
            pl.BlockSpec((RT, E), lambda t: (t, 0)),
            pl.BlockSpec((RT, DHW), lambda t: (t, 0)),
            pl.BlockSpec((RT, DHW), lambda t: (t, 0)),
        ],
        out_shape=[
            jax.ShapeDtypeStruct((T, E), f32),
            jax.ShapeDtypeStruct((T, DHW), jnp.int32),
            jax.ShapeDtypeStruct((T, DHW), jnp.int32),
        ],
    )(hidden_states, gate_w)

    slo3, shi3, bexpf = pl.pallas_call(
        _slots_body,
        out_shape=[
            jax.ShapeDtypeStruct((SROWS, 128), jnp.int32),
            jax.ShapeDtypeStruct((SROWS, 128), jnp.int32),
            jax.ShapeDtypeStruct((NBP, 128), f32),
        ],
    )(aux)
    slo1 = slo3.reshape(T)
    shi1 = shi3.reshape(T)
    bexp = bexpf[:, 0].astype(jnp.int32)

    mesh = plsc.VectorSubcoreMesh(core_axis_name="c", subcore_axis_name="s")

    dispatch = pl.kernel(
        _dispatch_body,
        out_type=[
            jax.ShapeDtypeStruct((P, DHW), jnp.int32),
            jax.ShapeDtypeStruct((P, DHW), jnp.int32),
        ],
        mesh=mesh,
        scratch_types=[
            pltpu.VMEM((128,), jnp.int32),
            pltpu.VMEM((128, DHW), jnp.int32),
        ],
    )
    xsa3, xsb3 = dispatch(xa, xb, slo1, shi1)

    gup_bf = gate_up_w.astype(bf16)

    ya, yb = pl.pallas_call(
        _mlp_body,
        grid_spec=pltpu.PrefetchScalarGridSpec(
            num_scalar_prefetch=1,
            grid=(NB,),
            in_specs=[
                pl.BlockSpec((BS, DHW), lambda b, be: (b, 0)),
                pl.BlockSpec((BS, DHW), lambda b, be: (b, 0)),
                pl.BlockSpec((1, D, 2 * DFF), lambda b, be: (be[b], 0, 0)),
                pl.BlockSpec((1, DFF, D), lambda b, be: (be[b], 0, 0)),
            ],
            out_specs=[
                pl.BlockSpec((BS, DHW), lambda b, be: (b, 0)),
                pl.BlockSpec((BS, DHW), lambda b, be: (b, 0)),
            ],
        ),
        out_shape=[
            jax.ShapeDtypeStruct((P, DHW), jnp.int32),
            jax.ShapeDtypeStruct((P, DHW), jnp.int32),
        ],
        compiler_params=pltpu.CompilerParams(
            dimension_semantics=("arbitrary",),
            vmem_limit_bytes=100 * 1024 * 1024,
        ),
    )(bexp, xsa3, xsb3, gup_bf, down_w)

    gather = pl.kernel(
        _gather_body,
        out_type=[
            jax.ShapeDtypeStruct((T, DHW), jnp.int32),
            jax.ShapeDtypeStruct((T, DHW), jnp.int32),
            jax.ShapeDtypeStruct((T, DHW), jnp.int32),
            jax.ShapeDtypeStruct((T, DHW), jnp.int32),
        ],
        mesh=mesh,
        scratch_types=(
            [pltpu.VMEM((64,), jnp.int32) for _ in range(8)]
            + [pltpu.VMEM((64, DHW), jnp.int32) for _ in range(2)]
            + [pltpu.SemaphoreType.DMA for _ in range(4)]
        ),
    )
    la3, lb3, ha3, hb3 = gather(ya, yb, slo1, shi1)

    out = pl.pallas_call(
        _combine_body,
        grid=(T // RT,),
        in_specs=[
            pl.BlockSpec((RT, E), lambda t: (t, 0)),
            pl.BlockSpec((RT, DHW), lambda t: (t, 0)),
            pl.BlockSpec((RT, DHW), lambda t: (t, 0)),
            pl.BlockSpec((RT, DHW), lambda t: (t, 0)),
            pl.BlockSpec((RT, DHW), lambda t: (t, 0)),
        ],
        out_specs=pl.BlockSpec((RT, D), lambda t: (t, 0)),
        out_shape=jax.ShapeDtypeStruct((T, D), f32),
    )(aux, la3, lb3, ha3, hb3)
    return out


# speedup vs baseline: 1.0081x; 1.0003x over previous
import jax
import jax.numpy as jnp
from jax import lax
from jax.experimental import pallas as pl
from jax.experimental.pallas import tpu as pltpu
from jax.experimental.pallas import tpu_sc as plsc

E = 8
D = 2048
DH = D // 2
DHW = DH // 2
DFF = 1408
T = 8192
TK = 2 * T
BS = 512
NB = TK // BS + E
NBP = NB + 8
P = NB * BS
RT = 1024

NC = 2
NS = 16
NW = NC * NS
SROWS = 64
WROWS = SROWS // NW


def _pack_pair(xf32):
    h = xf32.shape[1] // 2
    lo = lax.bitcast_convert_type(
        xf32[:, :h].astype(jnp.bfloat16).astype(jnp.float32), jnp.int32)
    hi = lax.bitcast_convert_type(
        xf32[:, h:].astype(jnp.bfloat16).astype(jnp.float32), jnp.int32)
    return lax.shift_right_logical(lo, 16) | (hi & (-65536))


def _unpack2(w):
    lo = lax.bitcast_convert_type(lax.shift_left(w, 16), jnp.float32)
    hi = lax.bitcast_convert_type(w & (-65536), jnp.float32)
    return lo, hi


def _router_body(x_ref, gw_ref, aux_ref, xa_ref, xb_ref):
    x = x_ref[...]
    logits = lax.dot_general(x, gw_ref[...], (((1,), (0,)), ((), ())),
                             preferred_element_type=jnp.float32)
    p = jax.nn.softmax(logits, axis=-1)
    lane = lax.broadcasted_iota(jnp.int32, p.shape, 1)
    m1 = jnp.max(p, axis=-1, keepdims=True)
    idx1 = jnp.min(jnp.where(p == m1, lane, E), axis=-1, keepdims=True)
    pex = jnp.where(lane == idx1, -1.0, p)
    m2 = jnp.max(pex, axis=-1, keepdims=True)
    idx2 = jnp.min(jnp.where(pex == m2, lane, E), axis=-1, keepdims=True)
    denom = m1 + m2
    swap = idx2 < idx1
    elo = jnp.where(swap, idx2, idx1).astype(jnp.float32)
    ehi = jnp.where(swap, idx1, idx2).astype(jnp.float32)
    wlo = jnp.where(swap, m2, m1) / denom
    whi = jnp.where(swap, m1, m2) / denom
    aux_ref[...] = (jnp.where(lane == 0, elo, 0.0)
                    + jnp.where(lane == 1, ehi, 0.0)
                    + jnp.where(lane == 2, wlo, 0.0)
                    + jnp.where(lane == 3, whi, 0.0))
    xa_ref[...] = _pack_pair(x[:, :DH])
    xb_ref[...] = _pack_pair(x[:, DH:])


def _slots_body(aux_ref, slo_ref, shi_ref, bexp_ref):
    aux = aux_ref[...]
    lane = lax.broadcasted_iota(jnp.int32, (T, E), 1)
    oh_lo = (lane == aux[:, 0:1].astype(jnp.int32)).astype(jnp.float32)
    oh_hi = (lane == aux[:, 1:2].astype(jnp.int32)).astype(jnp.float32)
    v_lo = oh_lo.reshape(SROWS, 128, E)
    v_hi = oh_hi.reshape(SROWS, 128, E)

    r_i = lax.broadcasted_iota(jnp.int32, (SROWS, 128, 128), 1)
    r_j = lax.broadcasted_iota(jnp.int32, (SROWS, 128, 128), 2)
    ltri = (r_j < r_i).astype(jnp.float32)

    def pancum(v):
        return lax.dot_general(ltri, v, (((2,), (1,)), ((0,), (0,))),
                               preferred_element_type=jnp.float32)

    w_lo, w_hi = pancum(v_lo), pancum(v_hi)
    s_lo = jnp.sum(v_lo, axis=1)
    s_hi = jnp.sum(v_hi, axis=1)
    p_i = lax.broadcasted_iota(jnp.int32, (SROWS, SROWS), 0)
    p_j = lax.broadcasted_iota(jnp.int32, (SROWS, SROWS), 1)
    l64 = (p_j < p_i).astype(jnp.float32)
    off_lo = lax.dot_general(l64, s_lo, (((1,), (0,)), ((), ())),
                             preferred_element_type=jnp.float32)
    off_hi = lax.dot_general(l64, s_hi, (((1,), (0,)), ((), ())),
                             preferred_element_type=jnp.float32)
    tot_lo = jnp.sum(s_lo, axis=0, keepdims=True)
    tot_hi = jnp.sum(s_hi, axis=0, keepdims=True)

    counts = (tot_lo + tot_hi).astype(jnp.int32)
    padded = (((counts + BS - 1) // BS) * BS).astype(jnp.float32)
    e_i = lax.broadcasted_iota(jnp.int32, (E, E), 0)
    e_j = lax.broadcasted_iota(jnp.int32, (E, E), 1)
    m8 = (e_i < e_j).astype(jnp.float32)
    gstart = lax.dot_general(padded, m8, (((1,), (0,)), ((), ())),
                             preferred_element_type=jnp.float32)

    c_lo = w_lo + off_lo.reshape(SROWS, 1, E)
    c_hi = w_hi + off_hi.reshape(SROWS, 1, E) + tot_lo.reshape(1, 1, E)
    base = gstart.reshape(1, 1, E)
    slot_lo = jnp.sum((c_lo + base) * v_lo, axis=2)
    slot_hi = jnp.sum((c_hi + base) * v_hi, axis=2)
    slo_ref[...] = slot_lo.astype(jnp.int32)
    shi_ref[...] = slot_hi.astype(jnp.int32)

    tot_pad = jnp.sum(padded, axis=1, keepdims=True)
    b_i = (lax.broadcasted_iota(jnp.int32, (NBP, E), 0) * BS).astype(jnp.float32)
    b_c = jnp.minimum(b_i, tot_pad - BS)
    nle = jnp.sum((gstart <= b_c).astype(jnp.float32), axis=1,
                  keepdims=True) - 1.0
    row = lax.broadcasted_iota(jnp.int32, (NBP, 1), 0)
    val = jnp.where(row < NB, nle, tot_pad / BS)
    bexp_ref[...] = jnp.broadcast_to(val, (NBP, 128))


def _dispatch_body(xa_hbm, xb_hbm, slo_hbm, shi_hbm, oa_hbm, ob_hbm,
                   slot_v, buf_v):
    wid = lax.axis_index("s") * NC + lax.axis_index("c")
    for slot_hbm in (slo_hbm, shi_hbm):
        for r in range(WROWS):
            tok0 = (wid * WROWS + r) * 128
            pltpu.sync_copy(slot_hbm.at[pl.ds(tok0, 128)], slot_v)
            pltpu.sync_copy(xa_hbm.at[pl.ds(tok0, 128)], buf_v)
            pltpu.sync_copy(buf_v, oa_hbm.at[slot_v])
            pltpu.sync_copy(xb_hbm.at[pl.ds(tok0, 128)], buf_v)
            pltpu.sync_copy(buf_v, ob_hbm.at[slot_v])


def _mlp_body(bexp_ref, xa_ref, xb_ref, gup_ref, dwn_ref, ya_ref, yb_ref):
    @pl.when(pl.program_id(0) < bexp_ref[NB])
    def _():
        _mlp_compute(xa_ref, xb_ref, gup_ref, dwn_ref, ya_ref, yb_ref)


def _mlp_compute(xa_ref, xb_ref, gup_ref, dwn_ref, ya_ref, yb_ref):
    la, ha = _unpack2(xa_ref[...])
    lb, hb = _unpack2(xb_ref[...])
    x = jnp.concatenate([la, ha, lb, hb], axis=1).astype(jnp.bfloat16)
    gg = lax.dot_general(x, gup_ref[0, :, :DFF], (((1,), (0,)), ((), ())),
                         preferred_element_type=jnp.float32)
    gu = lax.dot_general(x, gup_ref[0, :, DFF:], (((1,), (0,)), ((), ())),
                         preferred_element_type=jnp.float32)
    a = (jax.nn.silu(gg) * gu).astype(jnp.bfloat16)
    y = lax.dot_general(a, dwn_ref[0].astype(jnp.bfloat16),
                        (((1,), (0,)), ((), ())),
                        preferred_element_type=jnp.float32)
    ya_ref[...] = _pack_pair(y[:, :DH])
    yb_ref[...] = _pack_pair(y[:, DH:])


def _gather_body(ya_hbm, yb_hbm, slo_hbm, shi_hbm,
                 la_hbm, lb_hbm, ha_hbm, hb_hbm,
                 i0, i1, i2, i3, i4, i5, i6, i7,
                 buf0, buf1, sg0, sg1, sw0, sw1):
    wid = lax.axis_index("s") * NC + lax.axis_index("c")
    idx_refs = (i0, i1, i2, i3, i4, i5, i6, i7)
    items = []
    k = 0
    for slot_hbm, da_hbm, db_hbm in ((slo_hbm, la_hbm, lb_hbm),
                                     (shi_hbm, ha_hbm, hb_hbm)):
        for r in range(WROWS):
            for c in range(2):
                tok = (wid * WROWS + r) * 128 + c * 64
                pltpu.sync_copy(slot_hbm.at[pl.ds(tok, 64)], idx_refs[k])
                items.append((idx_refs[k], ya_hbm, da_hbm, tok))
                items.append((idx_refs[k], yb_hbm, db_hbm, tok))
                k += 1
    bufs, sgs, sws = (buf0, buf1), (sg0, sg1), (sw0, sw1)
    gops = [None, None]
    wops = [None, None]
    n = len(items)
    for i in range(n + 1):
        if i < n:
            if i >= 2:
                wops[i % 2].wait()
            idx, src, _, _ = items[i]
            gops[i % 2] = pltpu.make_async_copy(src.at[idx], bufs[i % 2],
                                                sgs[i % 2])
            gops[i % 2].start()
        if i >= 1:
            j = i - 1
            gops[j % 2].wait()
            _, _, dst, tok = items[j]
            wops[j % 2] = pltpu.make_async_copy(
                bufs[j % 2], dst.at[pl.ds(tok, 64)], sws[j % 2])
            wops[j % 2].start()
    wops[(n - 1) % 2].wait()
    wops[(n - 2) % 2].wait()


def _combine_body(aux_ref, la_ref, lb_ref, ha_ref, hb_ref, out_ref):
    aux = aux_ref[...]
    lane = lax.broadcasted_iota(jnp.int32, aux.shape, 1)
    wlo = jnp.sum(jnp.where(lane == 2, aux, 0.0), axis=1, keepdims=True)
    whi = jnp.sum(jnp.where(lane == 3, aux, 0.0), axis=1, keepdims=True)
    l0, l1 = _unpack2(la_ref[...])
    l2, l3 = _unpack2(lb_ref[...])
    h0, h1 = _unpack2(ha_ref[...])
    h2, h3 = _unpack2(hb_ref[...])
    for i, (lv, hv) in enumerate(((l0, h0), (l1, h1), (l2, h2), (l3, h3))):
        out_ref[:, i * DHW:(i + 1) * DHW] = wlo * lv + whi * hv


def kernel(hidden_states, gate_w, gate_up_w, down_w):
    f32 = jnp.float32
    bf16 = jnp.bfloat16

    aux, xa, xb = pl.pallas_call(
        _router_body,
        grid=(T // RT,),
        in_specs=[
            pl.BlockSpec((RT, D), lambda t: (t, 0)),
            pl.BlockSpec((D, E), lambda t: (0, 0)),
        ],
        out_specs=[
            pl.BlockSpec((RT, E), lambda t: (t, 0)),
            pl.BlockSpec((RT, DHW), lambda t: (t, 0)),
            pl.BlockSpec((RT, DHW), lambda t: (t, 0)),
        ],
        out_shape=[
            jax.ShapeDtypeStruct((T, E), f32),
            jax.ShapeDtypeStruct((T, DHW), jnp.int32),
            jax.ShapeDtypeStruct((T, DHW), jnp.int32),
        ],
    )(hidden_states, gate_w)

    slo3, shi3, bexpf = pl.pallas_call(
        _slots_body,
        out_shape=[
            jax.ShapeDtypeStruct((SROWS, 128), jnp.int32),
            jax.ShapeDtypeStruct((SROWS, 128), jnp.int32),
            jax.ShapeDtypeStruct((NBP, 128), f32),
        ],
    )(aux)
    slo1 = slo3.reshape(T)
    shi1 = shi3.reshape(T)
    bexp = bexpf[:, 0].astype(jnp.int32)

    mesh = plsc.VectorSubcoreMesh(core_axis_name="c", subcore_axis_name="s")

    dispatch = pl.kernel(
        _dispatch_body,
        out_type=[
            jax.ShapeDtypeStruct((P, DHW), jnp.int32),
            jax.ShapeDtypeStruct((P, DHW), jnp.int32),
        ],
        mesh=mesh,
        scratch_types=[
            pltpu.VMEM((128,), jnp.int32),
            pltpu.VMEM((128, DHW), jnp.int32),
        ],
    )
    xsa3, xsb3 = dispatch(xa, xb, slo1, shi1)

    gup_bf = gate_up_w.astype(bf16)

    ya, yb = pl.pallas_call(
        _mlp_body,
        grid_spec=pltpu.PrefetchScalarGridSpec(
            num_scalar_prefetch=1,
            grid=(NB,),
            in_specs=[
                pl.BlockSpec((BS, DHW), lambda b, be: (b, 0)),
                pl.BlockSpec((BS, DHW), lambda b, be: (b, 0)),
                pl.BlockSpec((1, D, 2 * DFF), lambda b, be: (be[b], 0, 0)),
                pl.BlockSpec((1, DFF, D), lambda b, be: (be[b], 0, 0)),
            ],
            out_specs=[
                pl.BlockSpec((BS, DHW), lambda b, be: (b, 0)),
                pl.BlockSpec((BS, DHW), lambda b, be: (b, 0)),
            ],
        ),
        out_shape=[
            jax.ShapeDtypeStruct((P, DHW), jnp.int32),
            jax.ShapeDtypeStruct((P, DHW), jnp.int32),
        ],
        compiler_params=pltpu.CompilerParams(
            dimension_semantics=("arbitrary",),
            vmem_limit_bytes=100 * 1024 * 1024,
        ),
    )(bexp, xsa3, xsb3, gup_bf, down_w)

    gather = pl.kernel(
        _gather_body,
        out_type=[
            jax.ShapeDtypeStruct((T, DHW), jnp.int32),
            jax.ShapeDtypeStruct((T, DHW), jnp.int32),
            jax.ShapeDtypeStruct((T, DHW), jnp.int32),
            jax.ShapeDtypeStruct((T, DHW), jnp.int32),
        ],
        mesh=mesh,
        scratch_types=(
            [pltpu.VMEM((64,), jnp.int32) for _ in range(8)]
            + [pltpu.VMEM((64, DHW), jnp.int32) for _ in range(2)]
            + [pltpu.SemaphoreType.DMA for _ in range(4)]
        ),
    )
    la3, lb3, ha3, hb3 = gather(ya, yb, slo1, shi1)

    out = pl.pallas_call(
        _combine_body,
        grid=(T // RT,),
        in_specs=[
            pl.BlockSpec((RT, E), lambda t: (t, 0)),
            pl.BlockSpec((RT, DHW), lambda t: (t, 0)),
            pl.BlockSpec((RT, DHW), lambda t: (t, 0)),
            pl.BlockSpec((RT, DHW), lambda t: (t, 0)),
            pl.BlockSpec((RT, DHW), lambda t: (t, 0)),
        ],
        out_specs=pl.BlockSpec((RT, D), lambda t: (t, 0)),
        out_shape=jax.ShapeDtypeStruct((T, D), f32),
    )(aux, la3, lb3, ha3, hb3)
    return out


# speedup vs baseline: 1.0085x; 1.0003x over previous
import jax
import jax.numpy as jnp
from jax import lax
from jax.experimental import pallas as pl
from jax.experimental.pallas import tpu as pltpu
from jax.experimental.pallas import tpu_sc as plsc

E = 8
D = 2048
DH = D // 2
DHW = DH // 2
DFF = 1408
T = 8192
TK = 2 * T
BS = 512
NB = TK // BS + E
NBP = NB + 8
P = NB * BS
RT = 1024

NC = 2
NS = 16
NW = NC * NS
SROWS = 64
WROWS = SROWS // NW


def _pack_pair(xf32):
    h = xf32.shape[1] // 2
    lo = lax.bitcast_convert_type(
        xf32[:, :h].astype(jnp.bfloat16).astype(jnp.float32), jnp.int32)
    hi = lax.bitcast_convert_type(
        xf32[:, h:].astype(jnp.bfloat16).astype(jnp.float32), jnp.int32)
    return lax.shift_right_logical(lo, 16) | (hi & (-65536))


def _unpack2(w):
    lo = lax.bitcast_convert_type(lax.shift_left(w, 16), jnp.float32)
    hi = lax.bitcast_convert_type(w & (-65536), jnp.float32)
    return lo, hi


def _router_body(x_ref, gw_ref, aux_ref, xa_ref, xb_ref):
    x = x_ref[...]
    logits = lax.dot_general(x, gw_ref[...], (((1,), (0,)), ((), ())),
                             preferred_element_type=jnp.float32)
    p = jax.nn.softmax(logits, axis=-1)
    lane = lax.broadcasted_iota(jnp.int32, p.shape, 1)
    m1 = jnp.max(p, axis=-1, keepdims=True)
    idx1 = jnp.min(jnp.where(p == m1, lane, E), axis=-1, keepdims=True)
    pex = jnp.where(lane == idx1, -1.0, p)
    m2 = jnp.max(pex, axis=-1, keepdims=True)
    idx2 = jnp.min(jnp.where(pex == m2, lane, E), axis=-1, keepdims=True)
    denom = m1 + m2
    swap = idx2 < idx1
    elo = jnp.where(swap, idx2, idx1).astype(jnp.float32)
    ehi = jnp.where(swap, idx1, idx2).astype(jnp.float32)
    wlo = jnp.where(swap, m2, m1) / denom
    whi = jnp.where(swap, m1, m2) / denom
    aux_ref[...] = (jnp.where(lane == 0, elo, 0.0)
                    + jnp.where(lane == 1, ehi, 0.0)
                    + jnp.where(lane == 2, wlo, 0.0)
                    + jnp.where(lane == 3, whi, 0.0))
    xa_ref[...] = _pack_pair(x[:, :DH])
    xb_ref[...] = _pack_pair(x[:, DH:])


def _slots_body(aux_ref, slo_ref, shi_ref, bexp_ref):
    aux = aux_ref[...]
    lane = lax.broadcasted_iota(jnp.int32, (T, E), 1)
    oh_lo = (lane == aux[:, 0:1].astype(jnp.int32)).astype(jnp.float32)
    oh_hi = (lane == aux[:, 1:2].astype(jnp.int32)).astype(jnp.float32)
    v_lo = oh_lo.reshape(SROWS, 128, E)
    v_hi = oh_hi.reshape(SROWS, 128, E)

    r_i = lax.broadcasted_iota(jnp.int32, (SROWS, 128, 128), 1)
    r_j = lax.broadcasted_iota(jnp.int32, (SROWS, 128, 128), 2)
    ltri = (r_j < r_i).astype(jnp.float32)

    def pancum(v):
        return lax.dot_general(ltri, v, (((2,), (1,)), ((0,), (0,))),
                               preferred_element_type=jnp.float32)

    w_lo, w_hi = pancum(v_lo), pancum(v_hi)
    s_lo = jnp.sum(v_lo, axis=1)
    s_hi = jnp.sum(v_hi, axis=1)
    p_i = lax.broadcasted_iota(jnp.int32, (SROWS, SROWS), 0)
    p_j = lax.broadcasted_iota(jnp.int32, (SROWS, SROWS), 1)
    l64 = (p_j < p_i).astype(jnp.float32)
    off_lo = lax.dot_general(l64, s_lo, (((1,), (0,)), ((), ())),
                             preferred_element_type=jnp.float32)
    off_hi = lax.dot_general(l64, s_hi, (((1,), (0,)), ((), ())),
                             preferred_element_type=jnp.float32)
    tot_lo = jnp.sum(s_lo, axis=0, keepdims=True)
    tot_hi = jnp.sum(s_hi, axis=0, keepdims=True)

    counts = (tot_lo + tot_hi).astype(jnp.int32)
    padded = (((counts + BS - 1) // BS) * BS).astype(jnp.float32)
    e_i = lax.broadcasted_iota(jnp.int32, (E, E), 0)
    e_j = lax.broadcasted_iota(jnp.int32, (E, E), 1)
    m8 = (e_i < e_j).astype(jnp.float32)
    gstart = lax.dot_general(padded, m8, (((1,), (0,)), ((), ())),
                             preferred_element_type=jnp.float32)

    c_lo = w_lo + off_lo.reshape(SROWS, 1, E)
    c_hi = w_hi + off_hi.reshape(SROWS, 1, E) + tot_lo.reshape(1, 1, E)
    base = gstart.reshape(1, 1, E)
    slot_lo = jnp.sum((c_lo + base) * v_lo, axis=2)
    slot_hi = jnp.sum((c_hi + base) * v_hi, axis=2)
    slo_ref[...] = slot_lo.astype(jnp.int32)
    shi_ref[...] = slot_hi.astype(jnp.int32)

    tot_pad = jnp.sum(padded, axis=1, keepdims=True)
    b_i = (lax.broadcasted_iota(jnp.int32, (NBP, E), 0) * BS).astype(jnp.float32)
    b_c = jnp.minimum(b_i, tot_pad - BS)
    nle = jnp.sum((gstart <= b_c).astype(jnp.float32), axis=1,
                  keepdims=True) - 1.0
    row = lax.broadcasted_iota(jnp.int32, (NBP, 1), 0)
    val = jnp.where(row < NB, nle, tot_pad / BS)
    bexp_ref[...] = jnp.broadcast_to(val, (NBP, 128))


def _dispatch_body(xa_hbm, xb_hbm, slo_hbm, shi_hbm, oa_hbm, ob_hbm,
                   slot_v, buf_v):
    wid = lax.axis_index("s") * NC + lax.axis_index("c")
    for slot_hbm in (slo_hbm, shi_hbm):
        for r in range(WROWS):
            tok0 = (wid * WROWS + r) * 128
            pltpu.sync_copy(slot_hbm.at[pl.ds(tok0, 128)], slot_v)
            pltpu.sync_copy(xa_hbm.at[pl.ds(tok0, 128)], buf_v)
            pltpu.sync_copy(buf_v, oa_hbm.at[slot_v])
            pltpu.sync_copy(xb_hbm.at[pl.ds(tok0, 128)], buf_v)
            pltpu.sync_copy(buf_v, ob_hbm.at[slot_v])


def _mlp_body(bexp_ref, xa_ref, xb_ref, gup_ref, dwn_ref, ya_ref, yb_ref):
    @pl.when(pl.program_id(0) < bexp_ref[NB])
    def _():
        _mlp_compute(xa_ref, xb_ref, gup_ref, dwn_ref, ya_ref, yb_ref)


def _mlp_compute(xa_ref, xb_ref, gup_ref, dwn_ref, ya_ref, yb_ref):
    la, ha = _unpack2(xa_ref[...])
    lb, hb = _unpack2(xb_ref[...])
    b16 = jnp.bfloat16
    x = jnp.concatenate([la.astype(b16), ha.astype(b16),
                         lb.astype(b16), hb.astype(b16)], axis=1)
    gg = lax.dot_general(x, gup_ref[0, :, :DFF], (((1,), (0,)), ((), ())),
                         preferred_element_type=jnp.float32)
    gu = lax.dot_general(x, gup_ref[0, :, DFF:], (((1,), (0,)), ((), ())),
                         preferred_element_type=jnp.float32)
    a = (jax.nn.silu(gg) * gu).astype(jnp.bfloat16)
    y = lax.dot_general(a, dwn_ref[0].astype(jnp.bfloat16),
                        (((1,), (0,)), ((), ())),
                        preferred_element_type=jnp.float32)
    ya_ref[...] = _pack_pair(y[:, :DH])
    yb_ref[...] = _pack_pair(y[:, DH:])


def _gather_body(ya_hbm, yb_hbm, slo_hbm, shi_hbm,
                 la_hbm, lb_hbm, ha_hbm, hb_hbm,
                 i0, i1, i2, i3, i4, i5, i6, i7,
                 buf0, buf1, sg0, sg1, sw0, sw1):
    wid = lax.axis_index("s") * NC + lax.axis_index("c")
    idx_refs = (i0, i1, i2, i3, i4, i5, i6, i7)
    items = []
    k = 0
    for slot_hbm, da_hbm, db_hbm in ((slo_hbm, la_hbm, lb_hbm),
                                     (shi_hbm, ha_hbm, hb_hbm)):
        for r in range(WROWS):
            for c in range(2):
                tok = (wid * WROWS + r) * 128 + c * 64
                pltpu.sync_copy(slot_hbm.at[pl.ds(tok, 64)], idx_refs[k])
                items.append((idx_refs[k], ya_hbm, da_hbm, tok))
                items.append((idx_refs[k], yb_hbm, db_hbm, tok))
                k += 1
    bufs, sgs, sws = (buf0, buf1), (sg0, sg1), (sw0, sw1)
    gops = [None, None]
    wops = [None, None]
    n = len(items)
    for i in range(n + 1):
        if i < n:
            if i >= 2:
                wops[i % 2].wait()
            idx, src, _, _ = items[i]
            gops[i % 2] = pltpu.make_async_copy(src.at[idx], bufs[i % 2],
                                                sgs[i % 2])
            gops[i % 2].start()
        if i >= 1:
            j = i - 1
            gops[j % 2].wait()
            _, _, dst, tok = items[j]
            wops[j % 2] = pltpu.make_async_copy(
                bufs[j % 2], dst.at[pl.ds(tok, 64)], sws[j % 2])
            wops[j % 2].start()
    wops[(n - 1) % 2].wait()
    wops[(n - 2) % 2].wait()


def _combine_body(aux_ref, la_ref, lb_ref, ha_ref, hb_ref, out_ref):
    aux = aux_ref[...]
    lane = lax.broadcasted_iota(jnp.int32, aux.shape, 1)
    wlo = jnp.sum(jnp.where(lane == 2, aux, 0.0), axis=1, keepdims=True)
    whi = jnp.sum(jnp.where(lane == 3, aux, 0.0), axis=1, keepdims=True)
    l0, l1 = _unpack2(la_ref[...])
    l2, l3 = _unpack2(lb_ref[...])
    h0, h1 = _unpack2(ha_ref[...])
    h2, h3 = _unpack2(hb_ref[...])
    for i, (lv, hv) in enumerate(((l0, h0), (l1, h1), (l2, h2), (l3, h3))):
        out_ref[:, i * DHW:(i + 1) * DHW] = wlo * lv + whi * hv


def kernel(hidden_states, gate_w, gate_up_w, down_w):
    f32 = jnp.float32
    bf16 = jnp.bfloat16

    aux, xa, xb = pl.pallas_call(
        _router_body,
        grid=(T // RT,),
        in_specs=[
            pl.BlockSpec((RT, D), lambda t: (t, 0)),
            pl.BlockSpec((D, E), lambda t: (0, 0)),
        ],
        out_specs=[
            pl.BlockSpec((RT, E), lambda t: (t, 0)),
            pl.BlockSpec((RT, DHW), lambda t: (t, 0)),
            pl.BlockSpec((RT, DHW), lambda t: (t, 0)),
        ],
        out_shape=[
            jax.ShapeDtypeStruct((T, E), f32),
            jax.ShapeDtypeStruct((T, DHW), jnp.int32),
            jax.ShapeDtypeStruct((T, DHW), jnp.int32),
        ],
    )(hidden_states, gate_w)

    slo3, shi3, bexpf = pl.pallas_call(
        _slots_body,
        out_shape=[
            jax.ShapeDtypeStruct((SROWS, 128), jnp.int32),
            jax.ShapeDtypeStruct((SROWS, 128), jnp.int32),
            jax.ShapeDtypeStruct((NBP, 128), f32),
        ],
    )(aux)
    slo1 = slo3.reshape(T)
    shi1 = shi3.reshape(T)
    bexp = bexpf[:, 0].astype(jnp.int32)

    mesh = plsc.VectorSubcoreMesh(core_axis_name="c", subcore_axis_name="s")

    dispatch = pl.kernel(
        _dispatch_body,
        out_type=[
            jax.ShapeDtypeStruct((P, DHW), jnp.int32),
            jax.ShapeDtypeStruct((P, DHW), jnp.int32),
        ],
        mesh=mesh,
        scratch_types=[
            pltpu.VMEM((128,), jnp.int32),
            pltpu.VMEM((128, DHW), jnp.int32),
        ],
    )
    xsa3, xsb3 = dispatch(xa, xb, slo1, shi1)

    gup_bf = gate_up_w.astype(bf16)

    ya, yb = pl.pallas_call(
        _mlp_body,
        grid_spec=pltpu.PrefetchScalarGridSpec(
            num_scalar_prefetch=1,
            grid=(NB,),
            in_specs=[
                pl.BlockSpec((BS, DHW), lambda b, be: (b, 0)),
                pl.BlockSpec((BS, DHW), lambda b, be: (b, 0)),
                pl.BlockSpec((1, D, 2 * DFF), lambda b, be: (be[b], 0, 0)),
                pl.BlockSpec((1, DFF, D), lambda b, be: (be[b], 0, 0)),
            ],
            out_specs=[
                pl.BlockSpec((BS, DHW), lambda b, be: (b, 0)),
                pl.BlockSpec((BS, DHW), lambda b, be: (b, 0)),
            ],
        ),
        out_shape=[
            jax.ShapeDtypeStruct((P, DHW), jnp.int32),
            jax.ShapeDtypeStruct((P, DHW), jnp.int32),
        ],
        compiler_params=pltpu.CompilerParams(
            dimension_semantics=("arbitrary",),
            vmem_limit_bytes=100 * 1024 * 1024,
        ),
    )(bexp, xsa3, xsb3, gup_bf, down_w)

    gather = pl.kernel(
        _gather_body,
        out_type=[
            jax.ShapeDtypeStruct((T, DHW), jnp.int32),
            jax.ShapeDtypeStruct((T, DHW), jnp.int32),
            jax.ShapeDtypeStruct((T, DHW), jnp.int32),
            jax.ShapeDtypeStruct((T, DHW), jnp.int32),
        ],
        mesh=mesh,
        scratch_types=(
            [pltpu.VMEM((64,), jnp.int32) for _ in range(8)]
            + [pltpu.VMEM((64, DHW), jnp.int32) for _ in range(2)]
            + [pltpu.SemaphoreType.DMA for _ in range(4)]
        ),
    )
    la3, lb3, ha3, hb3 = gather(ya, yb, slo1, shi1)

    out = pl.pallas_call(
        _combine_body,
        grid=(T // RT,),
        in_specs=[
            pl.BlockSpec((RT, E), lambda t: (t, 0)),
            pl.BlockSpec((RT, DHW), lambda t: (t, 0)),
            pl.BlockSpec((RT, DHW), lambda t: (t, 0)),
            pl.BlockSpec((RT, DHW), lambda t: (t, 0)),
            pl.BlockSpec((RT, DHW), lambda t: (t, 0)),
        ],
        out_specs=pl.BlockSpec((RT, D), lambda t: (t, 0)),
        out_shape=jax.ShapeDtypeStruct((T, D), f32),
    )(aux, la3, lb3, ha3, hb3)
    return out
